# baseline (speedup 1.0000x reference)
import functools

import jax
import jax.numpy as jnp
from jax import lax
from jax.experimental import pallas as pl
from jax.experimental.pallas import tpu as pltpu
from jax.experimental.pallas import tpu_sc as plsc


def _lane_splat(vec, lane):
    idx = jnp.full((16, 1), lane, jnp.int32)
    dn = lax.GatherDimensionNumbers(
        offset_dims=(), collapsed_slice_dims=(0,), start_index_map=(0,))
    return lax.gather(vec, idx, dn, (1,),
                      mode=lax.GatherScatterMode.PROMISE_IN_BOUNDS)


_N = 10000
_E = 320000
_NC = 2
_NS = 16
_F = 128
_FH = 64

_K = 80
_NCH = _E // _NS // _K


def _make_spmm():
    K = _K
    n_chunks = _NCH
    WB_TILES = 10
    rows_per_tile = _N // WB_TILES

    mesh = plsc.VectorSubcoreMesh(core_axis_name="c", subcore_axis_name="s",
                                  num_cores=_NC, num_subcores=_NS)

    @functools.partial(
        pl.kernel,
        out_type=jax.ShapeDtypeStruct((_NC, _N, _F), jnp.float32),
        mesh=mesh,
        scratch_types=[
            pltpu.VMEM((K,), jnp.int32),
            pltpu.VMEM((K,), jnp.int32),
            pltpu.VMEM((K,), jnp.int32),
            pltpu.VMEM((K,), jnp.int32),
            pltpu.VMEM((K,), jnp.float32),
            pltpu.VMEM((K,), jnp.float32),
            pltpu.VMEM((K, _F), jnp.float32),
            pltpu.VMEM((K, _F), jnp.float32),
            pltpu.VMEM_SHARED((_N, _F), jnp.float32),
            pltpu.SemaphoreType.DMA,
            pltpu.SemaphoreType.DMA,
            pltpu.SemaphoreType.DMA,
            pltpu.SemaphoreType.DMA,
        ],
    )
    def spmm(src_hbm, dst_hbm, val_hbm, plane_hbm, out_hbm,
             src0, src1, dst0, dst1, val0, val1, rows0, rows1, tab,
             gs0, gs1, es0, es1):
        c = lax.axis_index("c")
        s = lax.axis_index("s")

        @pl.when(s < WB_TILES)
        def _():
            r0 = pl.multiple_of(s * rows_per_tile, 8)
            pltpu.sync_copy(plane_hbm.at[c, pl.ds(r0, rows_per_tile)],
                            tab.at[pl.ds(r0, rows_per_tile)])
        plsc.subcore_barrier()

        def edge_start(j, sb, db, vb, sem):
            pltpu.async_copy(src_hbm.at[s, j], sb, sem)
            pltpu.async_copy(dst_hbm.at[s, j], db, sem)
            pltpu.async_copy(val_hbm.at[s, j], vb, sem)

        def edge_wait(j, sb, db, vb, sem):
            pltpu.make_async_copy(src_hbm.at[s, j], sb, sem).wait()
            pltpu.make_async_copy(dst_hbm.at[s, j], db, sem).wait()
            pltpu.make_async_copy(val_hbm.at[s, j], vb, sem).wait()

        def gather_start(sb, buf, sem):
            pltpu.async_copy(tab.at[sb], buf, sem)

        def gather_wait(sb, buf, sem):
            pltpu.make_async_copy(tab.at[sb], buf, sem).wait()

        def scale_scatter(db, vb, buf):
            def scale(g, _):
                valg = vb[pl.ds(16 * g, 16)]
                for l in range(16):
                    sp = _lane_splat(valg, l)
                    e = 16 * g + l
                    for f in range(_FH // 16):
                        lo = pl.ds(16 * f, 16)
                        hi = pl.ds(_FH + 16 * f, 16)
                        buf[e, hi] = buf[e, lo] * sp
                        buf[e, lo] = jnp.zeros((16,), jnp.float32)
                return 0
            lax.fori_loop(0, K // 16, scale, 0)
            pltpu.sync_copy(buf, tab.at[db], add=True)

        edge_start(0, src0, dst0, val0, es0)
        edge_start(1, src1, dst1, val1, es1)
        edge_wait(0, src0, dst0, val0, es0)
        gather_start(src0, rows0, gs0)

        def body(jj, _):
            j0 = 2 * jj
            j1 = j0 + 1
            not_last = jj < n_chunks // 2 - 1
            gather_wait(src0, rows0, gs0)
            edge_wait(j1, src1, dst1, val1, es1)
            gather_start(src1, rows1, gs1)
            scale_scatter(dst0, val0, rows0)

            @pl.when(not_last)
            def _():
                edge_start(j0 + 2, src0, dst0, val0, es0)
            gather_wait(src1, rows1, gs1)

            @pl.when(not_last)
            def _():
                edge_wait(j0 + 2, src0, dst0, val0, es0)
                gather_start(src0, rows0, gs0)
            scale_scatter(dst1, val1, rows1)

            @pl.when(not_last)
            def _():
                edge_start(j1 + 2, src1, dst1, val1, es1)
            return 0
        lax.fori_loop(0, n_chunks // 2, body, 0)

        plsc.subcore_barrier()

        @pl.when(s < WB_TILES)
        def _():
            r0 = pl.multiple_of(s * rows_per_tile, 8)
            pltpu.sync_copy(tab.at[pl.ds(r0, rows_per_tile)],
                            out_hbm.at[c, pl.ds(r0, rows_per_tile)])

    return spmm


_spmm = _make_spmm()


def _prep_edges(a):
    return a.reshape(_NS, _NCH, _K)


_BM = 1000


def _mm1_body(x_ref, w_ref, o_ref):
    t = jnp.dot(x_ref[...], w_ref[...], preferred_element_type=jnp.float32)
    z = jnp.zeros((t.shape[0], _FH), jnp.float32)
    o_ref[0] = jnp.concatenate([t[:, :_FH], z], axis=1)
    o_ref[1] = jnp.concatenate([t[:, _FH:], z], axis=1)


def _mm1(x, W1):
    M, Kd = x.shape
    return pl.pallas_call(
        _mm1_body,
        grid=(M // _BM,),
        in_specs=[
            pl.BlockSpec((_BM, Kd), lambda i: (i, 0)),
            pl.BlockSpec((Kd, _F), lambda i: (0, 0)),
        ],
        out_specs=pl.BlockSpec((2, _BM, _F), lambda i: (0, i, 0)),
        out_shape=jax.ShapeDtypeStruct((2, M, _F), jnp.float32),
    )(x, W1)


def _relu_split_body(p_ref, b_ref, o_ref):
    t = p_ref[:, :, _FH:] + b_ref[...][:, None, :]
    t = jnp.maximum(t, 0.0)
    z = jnp.zeros((2, t.shape[1], _FH), jnp.float32)
    o_ref[...] = jnp.concatenate([t, z], axis=2)


def _relu_split(p, b1):
    M = p.shape[1]
    return pl.pallas_call(
        _relu_split_body,
        grid=(M // _BM,),
        in_specs=[
            pl.BlockSpec((2, _BM, _F), lambda i: (0, i, 0)),
            pl.BlockSpec((2, _FH), lambda i: (0, 0)),
        ],
        out_specs=pl.BlockSpec((2, _BM, _F), lambda i: (0, i, 0)),
        out_shape=jax.ShapeDtypeStruct((2, M, _F), jnp.float32),
    )(p, b1.reshape(2, _FH))


def _mm2_body(g_ref, w_ref, b_ref, o_ref):
    t = jnp.concatenate([g_ref[0, :, _FH:], g_ref[1, :, _FH:]], axis=1)
    o_ref[...] = jnp.dot(t, w_ref[...],
                         preferred_element_type=jnp.float32) + b_ref[...]


def _mm2(g, W2, b2):
    M = g.shape[1]
    Nd = W2.shape[1]
    return pl.pallas_call(
        _mm2_body,
        grid=(M // _BM,),
        in_specs=[
            pl.BlockSpec((2, _BM, _F), lambda i: (0, i, 0)),
            pl.BlockSpec((_F, Nd), lambda i: (0, 0)),
            pl.BlockSpec((1, Nd), lambda i: (0, 0)),
        ],
        out_specs=pl.BlockSpec((_BM, Nd), lambda i: (i, 0)),
        out_shape=jax.ShapeDtypeStruct((M, Nd), jnp.float32),
    )(g, W2, b2.reshape(1, Nd))


def kernel(x, edge_index, adj_values, W1, b1, W2, b2):
    src = _prep_edges(edge_index[0])
    dst = _prep_edges(edge_index[1])
    adj_values = _prep_edges(adj_values)
    hp = _mm1(x, W1)
    p = _spmm(src, dst, adj_values, hp)
    zp = _relu_split(p, b1)
    g = _spmm(src, dst, adj_values, zp)
    return _mm2(g, W2, b2)


# speedup vs baseline: 2.2188x; 2.2188x over previous
import functools

import jax
import jax.numpy as jnp
from jax import lax
from jax.experimental import pallas as pl
from jax.experimental.pallas import tpu as pltpu
from jax.experimental.pallas import tpu_sc as plsc

def _lane_splat(vec, lane):
    idx = jnp.full((16, 1), lane, jnp.int32)
    dn = lax.GatherDimensionNumbers(
        offset_dims=(), collapsed_slice_dims=(0,), start_index_map=(0,))
    return lax.gather(vec, idx, dn, (1,),
                      mode=lax.GatherScatterMode.PROMISE_IN_BOUNDS)


_N = 10000
_E = 320000
_NC = 2
_NS = 16
_NW = _NC * _NS


_K = 80
_NCH = 125


def _make_spmm(F):
    K = _K
    n_chunks = _NCH
    WB_TILES = 10
    rows_per_tile = _N // WB_TILES
    FL = F // 16

    mesh = plsc.VectorSubcoreMesh(core_axis_name="c", subcore_axis_name="s",
                                  num_cores=_NC, num_subcores=_NS)

    @functools.partial(
        pl.kernel,
        out_type=jax.ShapeDtypeStruct((_NC, _N, F), jnp.float32),
        mesh=mesh,
        scratch_types=[
            pltpu.VMEM((n_chunks * K,), jnp.int32),
            pltpu.VMEM((n_chunks * K,), jnp.int32),
            pltpu.VMEM((n_chunks * K,), jnp.float32),
            pltpu.VMEM((K, F), jnp.float32),
            pltpu.VMEM((K, F), jnp.float32),
            pltpu.VMEM_SHARED((_N, F), jnp.float32),
            pltpu.SemaphoreType.DMA,
            pltpu.SemaphoreType.DMA,
            pltpu.SemaphoreType.DMA,
            pltpu.SemaphoreType.DMA,
        ],
    )
    def spmm(src_hbm, dst_hbm, val_hbm, h_hbm, zero_hbm, out_hbm,
             src_a, dst_a, val_a, rows0, rows1, acc, sem0, sem1, ss0, ss1):
        c = lax.axis_index("c")
        s = lax.axis_index("s")
        wid = s * _NC + c

        pltpu.sync_copy(src_hbm.at[wid], src_a)
        pltpu.sync_copy(dst_hbm.at[wid], dst_a)
        pltpu.sync_copy(val_hbm.at[wid], val_a)

        @pl.when(s < WB_TILES)
        def _():
            row0_ = pl.multiple_of(s * rows_per_tile, 8)
            pltpu.sync_copy(zero_hbm.at[pl.ds(row0_, rows_per_tile)],
                            acc.at[pl.ds(row0_, rows_per_tile)])
        plsc.subcore_barrier()

        def gather_start(j, buf, sem):
            pltpu.async_copy(h_hbm.at[src_a.at[pl.ds(j * K, K)]], buf, sem)

        def gather_wait(j, buf, sem):
            pltpu.make_async_copy(
                h_hbm.at[src_a.at[pl.ds(j * K, K)]], buf, sem).wait()

        def scale(j, buf):
            def scale_g(g, _):
                valg = val_a[pl.ds(j * K + 16 * g, 16)]
                for l in range(16):
                    vb = _lane_splat(valg, l)
                    e = 16 * g + l
                    for f in range(FL):
                        sl = pl.ds(16 * f, 16)
                        buf[e, sl] = buf[e, sl] * vb
                return 0
            lax.fori_loop(0, K // 16, scale_g, 0)

        def scatter_start(j, buf, sem):
            pltpu.async_copy(buf, acc.at[dst_a.at[pl.ds(j * K, K)]], sem,
                             add=True)

        def scatter_wait(j, buf, sem):
            pltpu.make_async_copy(
                buf, acc.at[dst_a.at[pl.ds(j * K, K)]], sem).wait()

        gather_start(0, rows0, sem0)

        def body(jj, _):
            j0 = 2 * jj
            j1 = j0 + 1

            @pl.when(jj > 0)
            def _():
                scatter_wait(j0 - 1, rows1, ss1)
            gather_start(j1, rows1, sem1)
            gather_wait(j0, rows0, sem0)
            scale(j0, rows0)
            scatter_start(j0, rows0, ss0)
            gather_wait(j1, rows1, sem1)
            scale(j1, rows1)
            scatter_wait(j0, rows0, ss0)
            gather_start(j0 + 2, rows0, sem0)
            scatter_start(j1, rows1, ss1)
            return 0
        lax.fori_loop(0, n_chunks // 2, body, 0)
        scatter_wait(n_chunks - 2, rows1, ss1)
        gather_wait(n_chunks - 1, rows0, sem0)
        scale(n_chunks - 1, rows0)
        pltpu.sync_copy(rows0, acc.at[dst_a.at[pl.ds((n_chunks - 1) * K, K)]],
                        add=True)

        plsc.subcore_barrier()

        @pl.when(s < WB_TILES)
        def _():
            row0_ = pl.multiple_of(s * rows_per_tile, 8)
            pltpu.sync_copy(
                acc.at[pl.ds(row0_, rows_per_tile)],
                out_hbm.at[c, pl.ds(row0_, rows_per_tile)],
            )

    return spmm


_spmm128 = _make_spmm(128)


def _prep_edges(a):
    return a.reshape(_NW, _E // _NW)


_BM = 1000


def _mm1_body(x_ref, w_ref, o_ref):
    o_ref[...] = jnp.dot(x_ref[...], w_ref[...],
                         preferred_element_type=jnp.float32)


def _mm1(x, W1):
    M, Kd = x.shape
    Nd = W1.shape[1]
    return pl.pallas_call(
        _mm1_body,
        grid=(M // _BM,),
        in_specs=[
            pl.BlockSpec((_BM, Kd), lambda i: (i, 0)),
            pl.BlockSpec((Kd, Nd), lambda i: (0, 0)),
        ],
        out_specs=pl.BlockSpec((_BM, Nd), lambda i: (i, 0)),
        out_shape=jax.ShapeDtypeStruct((M, Nd), jnp.float32),
    )(x, W1)


def _relu_merge_body(p_ref, b_ref, o_ref):
    o_ref[...] = jnp.maximum(p_ref[0] + p_ref[1] + b_ref[...], 0.0)


def _relu_merge(p, b1):
    M = p.shape[1]
    Kd = p.shape[2]
    return pl.pallas_call(
        _relu_merge_body,
        grid=(M // _BM,),
        in_specs=[
            pl.BlockSpec((2, _BM, Kd), lambda i: (0, i, 0)),
            pl.BlockSpec((1, Kd), lambda i: (0, 0)),
        ],
        out_specs=pl.BlockSpec((_BM, Kd), lambda i: (i, 0)),
        out_shape=jax.ShapeDtypeStruct((M, Kd), jnp.float32),
    )(p, b1.reshape(1, Kd))


def _mm2_body(g_ref, w_ref, b_ref, o_ref):
    t = g_ref[0] + g_ref[1]
    o_ref[...] = jnp.dot(t, w_ref[...],
                         preferred_element_type=jnp.float32) + b_ref[...]


def _mm2(g, W2, b2):
    M = g.shape[1]
    Kd = g.shape[2]
    Nd = W2.shape[1]
    return pl.pallas_call(
        _mm2_body,
        grid=(M // _BM,),
        in_specs=[
            pl.BlockSpec((2, _BM, Kd), lambda i: (0, i, 0)),
            pl.BlockSpec((Kd, Nd), lambda i: (0, 0)),
            pl.BlockSpec((1, Nd), lambda i: (0, 0)),
        ],
        out_specs=pl.BlockSpec((_BM, Nd), lambda i: (i, 0)),
        out_shape=jax.ShapeDtypeStruct((M, Nd), jnp.float32),
    )(g, W2, b2.reshape(1, Nd))


def kernel(x, edge_index, adj_values, W1, b1, W2, b2):
    src = _prep_edges(edge_index[0])
    dst = _prep_edges(edge_index[1])
    adj_values = _prep_edges(adj_values)
    zero = jnp.zeros((_N, 128), jnp.float32)
    h = _mm1(x, W1)
    p = _spmm128(src, dst, adj_values, h, zero)
    z = _relu_merge(p, b1)
    g = _spmm128(src, dst, adj_values, z, zero)
    return _mm2(g, W2, b2)


# speedup vs baseline: 2.3492x; 1.0588x over previous
import functools

import jax
import jax.numpy as jnp
from jax import lax
from jax.experimental import pallas as pl
from jax.experimental.pallas import tpu as pltpu
from jax.experimental.pallas import tpu_sc as plsc

def _lane_splat(vec, lane):
    idx = jnp.full((16, 1), lane, jnp.int32)
    dn = lax.GatherDimensionNumbers(
        offset_dims=(), collapsed_slice_dims=(0,), start_index_map=(0,))
    return lax.gather(vec, idx, dn, (1,),
                      mode=lax.GatherScatterMode.PROMISE_IN_BOUNDS)


_N = 10000
_E = 320000
_NC = 2
_NS = 16
_NW = _NC * _NS


_K = 80
_NCH = 125


def _make_spmm(F):
    K = _K
    n_chunks = _NCH
    T = _E // _NW
    WB_TILES = 10
    rows_per_tile = _N // WB_TILES
    FL = F // 16

    mesh = plsc.VectorSubcoreMesh(core_axis_name="c", subcore_axis_name="s",
                                  num_cores=_NC, num_subcores=_NS)

    @functools.partial(
        pl.kernel,
        out_type=jax.ShapeDtypeStruct((_NC, _N, F), jnp.float32),
        mesh=mesh,
        scratch_types=[
            pltpu.VMEM((n_chunks * K,), jnp.float32),
            pltpu.VMEM((K,), jnp.int32),
            pltpu.VMEM((K,), jnp.int32),
            pltpu.VMEM((K,), jnp.int32),
            pltpu.VMEM((K,), jnp.int32),
            pltpu.VMEM((K,), jnp.int32),
            pltpu.VMEM((K,), jnp.int32),
            pltpu.VMEM((K, F), jnp.float32),
            pltpu.VMEM((K, F), jnp.float32),
            pltpu.VMEM((K, F), jnp.float32),
            pltpu.VMEM_SHARED((_N, F), jnp.float32),
            pltpu.SemaphoreType.DMA,
            pltpu.SemaphoreType.DMA,
            pltpu.SemaphoreType.DMA,
            pltpu.SemaphoreType.DMA,
            pltpu.SemaphoreType.DMA,
            pltpu.SemaphoreType.DMA,
        ],
    )
    def spmm(src_hbm, dst_hbm, val_hbm, h_hbm, zero_hbm, out_hbm,
             val_a, srcb0, srcb1, srcb2, dstb0, dstb1, dstb2,
             rows0, rows1, rows2, acc,
             gsem0, gsem1, gsem2, esem0, esem1, esem2):
        c = lax.axis_index("c")
        s = lax.axis_index("s")
        wid = s * _NC + c

        pltpu.sync_copy(val_hbm.at[wid], val_a)

        @pl.when(s < WB_TILES)
        def _():
            row0_ = pl.multiple_of(s * rows_per_tile, 8)
            pltpu.sync_copy(zero_hbm.at[pl.ds(row0_, rows_per_tile)],
                            acc.at[pl.ds(row0_, rows_per_tile)])
        plsc.subcore_barrier()

        sbufs = (srcb0, srcb1, srcb2)
        dbufs = (dstb0, dstb1, dstb2)
        rbufs = (rows0, rows1, rows2)
        gsems = (gsem0, gsem1, gsem2)
        esems = (esem0, esem1, esem2)

        def edge_start(j, t):
            base = pl.multiple_of(wid * T + j * K, 8)
            pltpu.async_copy(src_hbm.at[pl.ds(base, K)], sbufs[t], esems[t])
            pltpu.async_copy(dst_hbm.at[pl.ds(base, K)], dbufs[t], esems[t])

        def edge_wait(j, t):
            base = pl.multiple_of(wid * T + j * K, 8)
            pltpu.make_async_copy(
                src_hbm.at[pl.ds(base, K)], sbufs[t], esems[t]).wait()
            pltpu.make_async_copy(
                dst_hbm.at[pl.ds(base, K)], dbufs[t], esems[t]).wait()

        def gather_start(t):
            pltpu.async_copy(h_hbm.at[sbufs[t]], rbufs[t], gsems[t])

        def gather_wait(t):
            pltpu.make_async_copy(h_hbm.at[sbufs[t]], rbufs[t],
                                  gsems[t]).wait()

        def scale_scatter(j, t):
            buf = rbufs[t]

            def scale(g, _):
                valg = val_a[pl.ds(j * K + 16 * g, 16)]
                for l in range(16):
                    vb = _lane_splat(valg, l)
                    e = 16 * g + l
                    for f in range(FL):
                        sl = pl.ds(16 * f, 16)
                        buf[e, sl] = buf[e, sl] * vb
                return 0
            lax.fori_loop(0, K // 16, scale, 0)
            pltpu.sync_copy(buf, acc.at[dbufs[t]], add=True)

        for t in range(3):
            edge_start(t, t)
        for t in range(2):
            edge_wait(t, t)
            gather_start(t)

        def body(jj, _):
            for t in range(3):
                j = 3 * jj + t
                gather_wait(t)
                scale_scatter(j, t)

                @pl.when(j + 3 < n_chunks)
                def _():
                    edge_start(j + 3, t)

                @pl.when(j + 2 < n_chunks)
                def _():
                    edge_wait(j + 2, (t + 2) % 3)
                    gather_start((t + 2) % 3)
            return 0
        lax.fori_loop(0, n_chunks // 3, body, 0)
        for t in range(2):
            j = (n_chunks // 3) * 3 + t
            gather_wait(t)
            scale_scatter(j, t)

        plsc.subcore_barrier()

        @pl.when(s < WB_TILES)
        def _():
            row0_ = pl.multiple_of(s * rows_per_tile, 8)
            pltpu.sync_copy(
                acc.at[pl.ds(row0_, rows_per_tile)],
                out_hbm.at[c, pl.ds(row0_, rows_per_tile)],
            )

    return spmm


_spmm128 = _make_spmm(128)


def _prep_val(a):
    return a.reshape(_NW, _E // _NW)


_BM = 1000


def _mm1_body(x_ref, w_ref, o_ref):
    o_ref[...] = jnp.dot(x_ref[...], w_ref[...],
                         preferred_element_type=jnp.float32)


def _mm1(x, W1):
    M, Kd = x.shape
    Nd = W1.shape[1]
    return pl.pallas_call(
        _mm1_body,
        grid=(M // _BM,),
        in_specs=[
            pl.BlockSpec((_BM, Kd), lambda i: (i, 0)),
            pl.BlockSpec((Kd, Nd), lambda i: (0, 0)),
        ],
        out_specs=pl.BlockSpec((_BM, Nd), lambda i: (i, 0)),
        out_shape=jax.ShapeDtypeStruct((M, Nd), jnp.float32),
    )(x, W1)


def _relu_merge_body(p_ref, b_ref, o_ref):
    o_ref[...] = jnp.maximum(p_ref[0] + p_ref[1] + b_ref[...], 0.0)


def _relu_merge(p, b1):
    M = p.shape[1]
    Kd = p.shape[2]
    return pl.pallas_call(
        _relu_merge_body,
        grid=(M // _BM,),
        in_specs=[
            pl.BlockSpec((2, _BM, Kd), lambda i: (0, i, 0)),
            pl.BlockSpec((1, Kd), lambda i: (0, 0)),
        ],
        out_specs=pl.BlockSpec((_BM, Kd), lambda i: (i, 0)),
        out_shape=jax.ShapeDtypeStruct((M, Kd), jnp.float32),
    )(p, b1.reshape(1, Kd))


def _mm2_body(g_ref, w_ref, b_ref, o_ref):
    t = g_ref[0] + g_ref[1]
    o_ref[...] = jnp.dot(t, w_ref[...],
                         preferred_element_type=jnp.float32) + b_ref[...]


def _mm2(g, W2, b2):
    M = g.shape[1]
    Kd = g.shape[2]
    Nd = W2.shape[1]
    return pl.pallas_call(
        _mm2_body,
        grid=(M // _BM,),
        in_specs=[
            pl.BlockSpec((2, _BM, Kd), lambda i: (0, i, 0)),
            pl.BlockSpec((Kd, Nd), lambda i: (0, 0)),
            pl.BlockSpec((1, Nd), lambda i: (0, 0)),
        ],
        out_specs=pl.BlockSpec((_BM, Nd), lambda i: (i, 0)),
        out_shape=jax.ShapeDtypeStruct((M, Nd), jnp.float32),
    )(g, W2, b2.reshape(1, Nd))


def kernel(x, edge_index, adj_values, W1, b1, W2, b2):
    src = edge_index[0]
    dst = edge_index[1]
    adj_values = _prep_val(adj_values)
    zero = jnp.zeros((_N, 128), jnp.float32)
    h = _mm1(x, W1)
    p = _spmm128(src, dst, adj_values, h, zero)
    z = _relu_merge(p, b1)
    g = _spmm128(src, dst, adj_values, z, zero)
    return _mm2(g, W2, b2)


# speedup vs baseline: 2.3725x; 1.0099x over previous
import functools

import jax
import jax.numpy as jnp
from jax import lax
from jax.experimental import pallas as pl
from jax.experimental.pallas import tpu as pltpu
from jax.experimental.pallas import tpu_sc as plsc

def _lane_splat(vec, lane):
    idx = jnp.full((16, 1), lane, jnp.int32)
    dn = lax.GatherDimensionNumbers(
        offset_dims=(), collapsed_slice_dims=(0,), start_index_map=(0,))
    return lax.gather(vec, idx, dn, (1,),
                      mode=lax.GatherScatterMode.PROMISE_IN_BOUNDS)


_N = 10000
_E = 320000
_NC = 2
_NS = 16
_NW = _NC * _NS


_K = 80
_NCH = 125


def _make_spmm(F):
    K = _K
    n_chunks = _NCH
    T = _E // _NW
    WB_TILES = 10
    rows_per_tile = _N // WB_TILES
    FL = F // 16

    mesh = plsc.VectorSubcoreMesh(core_axis_name="c", subcore_axis_name="s",
                                  num_cores=_NC, num_subcores=_NS)

    @functools.partial(
        pl.kernel,
        out_type=jax.ShapeDtypeStruct((_NC, _N, F), jnp.float32),
        mesh=mesh,
        scratch_types=[
            pltpu.VMEM((n_chunks * K,), jnp.float32),
            pltpu.VMEM((K,), jnp.int32),
            pltpu.VMEM((K,), jnp.int32),
            pltpu.VMEM((K,), jnp.int32),
            pltpu.VMEM((K,), jnp.int32),
            pltpu.VMEM((K,), jnp.int32),
            pltpu.VMEM((K,), jnp.int32),
            pltpu.VMEM((K, F), jnp.float32),
            pltpu.VMEM((K, F), jnp.float32),
            pltpu.VMEM((K, F), jnp.float32),
            pltpu.VMEM_SHARED((_N, F), jnp.float32),
            pltpu.SemaphoreType.DMA,
            pltpu.SemaphoreType.DMA,
            pltpu.SemaphoreType.DMA,
            pltpu.SemaphoreType.DMA,
            pltpu.SemaphoreType.DMA,
            pltpu.SemaphoreType.DMA,
        ],
    )
    def spmm(src_hbm, dst_hbm, val_hbm, h_hbm, out_hbm,
             val_a, srcb0, srcb1, srcb2, dstb0, dstb1, dstb2,
             rows0, rows1, rows2, acc,
             gsem0, gsem1, gsem2, esem0, esem1, esem2):
        c = lax.axis_index("c")
        s = lax.axis_index("s")
        wid = s * _NC + c

        pltpu.sync_copy(val_hbm.at[wid], val_a)

        @pl.when(s < WB_TILES)
        def _():
            def zrow(r, _):
                for f in range(FL):
                    rows0[r, pl.ds(16 * f, 16)] = jnp.zeros((16,), jnp.float32)
                return 0
            lax.fori_loop(0, K, zrow, 0)
            row0_ = pl.multiple_of(s * rows_per_tile, 8)
            for i in range(rows_per_tile // K):
                pltpu.sync_copy(rows0,
                                acc.at[pl.ds(row0_ + i * K, K)])
            rem = rows_per_tile % K
            if rem:
                pltpu.sync_copy(rows0.at[pl.ds(0, rem)],
                                acc.at[pl.ds(row0_ + (rows_per_tile // K) * K,
                                             rem)])
        plsc.subcore_barrier()

        sbufs = (srcb0, srcb1, srcb2)
        dbufs = (dstb0, dstb1, dstb2)
        rbufs = (rows0, rows1, rows2)
        gsems = (gsem0, gsem1, gsem2)
        esems = (esem0, esem1, esem2)

        def edge_start(j, t):
            base = pl.multiple_of(wid * T + j * K, 8)
            pltpu.async_copy(src_hbm.at[pl.ds(base, K)], sbufs[t], esems[t])
            pltpu.async_copy(dst_hbm.at[pl.ds(base, K)], dbufs[t], esems[t])

        def edge_wait(j, t):
            base = pl.multiple_of(wid * T + j * K, 8)
            pltpu.make_async_copy(
                src_hbm.at[pl.ds(base, K)], sbufs[t], esems[t]).wait()
            pltpu.make_async_copy(
                dst_hbm.at[pl.ds(base, K)], dbufs[t], esems[t]).wait()

        def gather_start(t):
            pltpu.async_copy(h_hbm.at[sbufs[t]], rbufs[t], gsems[t])

        def gather_wait(t):
            pltpu.make_async_copy(h_hbm.at[sbufs[t]], rbufs[t],
                                  gsems[t]).wait()

        def scale_scatter(j, t):
            buf = rbufs[t]

            def scale(g, _):
                valg = val_a[pl.ds(j * K + 16 * g, 16)]
                for l in range(16):
                    vb = _lane_splat(valg, l)
                    e = 16 * g + l
                    for f in range(FL):
                        sl = pl.ds(16 * f, 16)
                        buf[e, sl] = buf[e, sl] * vb
                return 0
            lax.fori_loop(0, K // 16, scale, 0)
            pltpu.sync_copy(buf, acc.at[dbufs[t]], add=True)

        for t in range(3):
            edge_start(t, t)
        for t in range(2):
            edge_wait(t, t)
            gather_start(t)

        def body(jj, _):
            for t in range(3):
                j = 3 * jj + t
                gather_wait(t)
                scale_scatter(j, t)

                @pl.when(j + 3 < n_chunks)
                def _():
                    edge_start(j + 3, t)

                @pl.when(j + 2 < n_chunks)
                def _():
                    edge_wait(j + 2, (t + 2) % 3)
                    gather_start((t + 2) % 3)
            return 0
        lax.fori_loop(0, n_chunks // 3, body, 0)
        for t in range(2):
            j = (n_chunks // 3) * 3 + t
            gather_wait(t)
            scale_scatter(j, t)

        plsc.subcore_barrier()

        @pl.when(s < WB_TILES)
        def _():
            row0_ = pl.multiple_of(s * rows_per_tile, 8)
            pltpu.sync_copy(
                acc.at[pl.ds(row0_, rows_per_tile)],
                out_hbm.at[c, pl.ds(row0_, rows_per_tile)],
            )

    return spmm


_spmm128 = _make_spmm(128)


def _prep_val(a):
    return a.reshape(_NW, _E // _NW)


_BM = 1000


def _mm1_body(x_ref, w_ref, o_ref):
    o_ref[...] = jnp.dot(x_ref[...], w_ref[...],
                         preferred_element_type=jnp.float32)


def _mm1(x, W1):
    M, Kd = x.shape
    Nd = W1.shape[1]
    return pl.pallas_call(
        _mm1_body,
        grid=(M // _BM,),
        in_specs=[
            pl.BlockSpec((_BM, Kd), lambda i: (i, 0)),
            pl.BlockSpec((Kd, Nd), lambda i: (0, 0)),
        ],
        out_specs=pl.BlockSpec((_BM, Nd), lambda i: (i, 0)),
        out_shape=jax.ShapeDtypeStruct((M, Nd), jnp.float32),
    )(x, W1)


def _relu_merge_body(p_ref, b_ref, o_ref):
    o_ref[...] = jnp.maximum(p_ref[0] + p_ref[1] + b_ref[...], 0.0)


def _relu_merge(p, b1):
    M = p.shape[1]
    Kd = p.shape[2]
    return pl.pallas_call(
        _relu_merge_body,
        grid=(M // _BM,),
        in_specs=[
            pl.BlockSpec((2, _BM, Kd), lambda i: (0, i, 0)),
            pl.BlockSpec((1, Kd), lambda i: (0, 0)),
        ],
        out_specs=pl.BlockSpec((_BM, Kd), lambda i: (i, 0)),
        out_shape=jax.ShapeDtypeStruct((M, Kd), jnp.float32),
    )(p, b1.reshape(1, Kd))


def _mm2_body(g_ref, w_ref, b_ref, o_ref):
    t = g_ref[0] + g_ref[1]
    o_ref[...] = jnp.dot(t, w_ref[...],
                         preferred_element_type=jnp.float32) + b_ref[...]


def _mm2(g, W2, b2):
    M = g.shape[1]
    Kd = g.shape[2]
    Nd = W2.shape[1]
    return pl.pallas_call(
        _mm2_body,
        grid=(M // _BM,),
        in_specs=[
            pl.BlockSpec((2, _BM, Kd), lambda i: (0, i, 0)),
            pl.BlockSpec((Kd, Nd), lambda i: (0, 0)),
            pl.BlockSpec((1, Nd), lambda i: (0, 0)),
        ],
        out_specs=pl.BlockSpec((_BM, Nd), lambda i: (i, 0)),
        out_shape=jax.ShapeDtypeStruct((M, Nd), jnp.float32),
    )(g, W2, b2.reshape(1, Nd))


def kernel(x, edge_index, adj_values, W1, b1, W2, b2):
    src = edge_index[0]
    dst = edge_index[1]
    adj_values = _prep_val(adj_values)
    h = _mm1(x, W1)
    p = _spmm128(src, dst, adj_values, h)
    z = _relu_merge(p, b1)
    g = _spmm128(src, dst, adj_values, z)
    return _mm2(g, W2, b2)
